# Initial kernel scaffold; baseline (speedup 1.0000x reference)
import jax
import jax.numpy as jnp
from jax.experimental import pallas as pl


def kernel(x, edge_index, edge_attr, W1_rel, b1_rel, W1_root, gamma1, beta1, W2_rel, b2_rel, W2_root, gamma2, beta2, W3_rel, b3_rel, W3_root):
    raise NotImplementedError("write your pallas kernel here")


# speedup vs baseline: 3.0549x; 3.0549x over previous
import functools

import jax
import jax.numpy as jnp
from jax import lax
from jax.experimental import pallas as pl
from jax.experimental.pallas import tpu as pltpu
from jax.experimental.pallas import tpu_sc as plsc

N = 10000
E = 320000
D = 128
HC2 = 256

NC = 2
NS = 16
NW = NC * NS
L = 16

CH = 128
NCHUNK = 80
EPT = NCHUNK * CH
E_PAD = NW * EPT
NPAD = 10240
RPT = NPAD // NS
RCP = 128
NRC = RPT // RCP


def _sc_agg(table, srcm, dstm, wm):
  mesh = plsc.VectorSubcoreMesh(core_axis_name="c", subcore_axis_name="s")

  @functools.partial(
      pl.kernel,
      mesh=mesh,
      out_type=jax.ShapeDtypeStruct((NC * NPAD, D), jnp.float32),
      scratch_types=[
          pltpu.VMEM((NCHUNK, CH), jnp.int32),
          pltpu.VMEM((NCHUNK, CH), jnp.int32),
          pltpu.VMEM((EPT,), jnp.float32),
          pltpu.VMEM((CH, D), jnp.float32),
          pltpu.VMEM_SHARED((NPAD, D), jnp.float32),
          pltpu.SemaphoreType.DMA,
      ],
  )
  def k(table_hbm, src_hbm, dst_hbm, w_hbm, out_hbm,
        src_v, dst_v, w_v, rows_v, acc, gsem):
    cid = lax.axis_index("c")
    sid = lax.axis_index("s")
    wid = sid * NC + cid

    pltpu.sync_copy(src_hbm.at[wid], src_v)
    pltpu.sync_copy(dst_hbm.at[wid], dst_v)
    pltpu.sync_copy(w_hbm.at[wid], w_v)

    zero = jnp.zeros((L,), jnp.float32)

    def zrow(r, carry):
      for g in range(D // L):
        rows_v[r, pl.ds(g * L, L)] = zero
      return carry

    lax.fori_loop(0, CH, zrow, 0)
    for kk in range(NRC):
      pltpu.sync_copy(rows_v.at[pl.ds(0, RCP)],
                      acc.at[pl.ds(sid * RPT + kk * RCP, RCP)])
    plsc.subcore_barrier()

    def chunk_body(j, carry):
      pltpu.async_copy(table_hbm.at[src_v.at[j]], rows_v, gsem).wait()

      def grp_body(g2, c2):
        wv = w_v[pl.ds(pl.multiple_of(j * CH + g2 * L, L), L)]
        dnums = lax.GatherDimensionNumbers(
            offset_dims=(), collapsed_slice_dims=(0,), start_index_map=(0,))
        for l in range(L):
          idx = lax.broadcast(jnp.int32(l), (L,))
          wsp = lax.gather(wv, idx[:, None], dnums, (1,),
                           mode=lax.GatherScatterMode.PROMISE_IN_BOUNDS)
          e = g2 * L + l
          for g in range(D // L):
            sl = pl.ds(g * L, L)
            rows_v[e, sl] = rows_v[e, sl] * wsp
        return c2

      lax.fori_loop(0, CH // L, grp_body, 0)
      pltpu.sync_copy(rows_v, acc.at[dst_v.at[j]], add=True)
      return carry

    lax.fori_loop(0, NCHUNK, chunk_body, 0)
    plsc.subcore_barrier()

    for kk in range(NRC):
      row0 = sid * RPT + kk * RCP
      pltpu.sync_copy(acc.at[pl.ds(row0, RCP)],
                      out_hbm.at[pl.ds(cid * NPAD + row0, RCP)])

  out = k(table, srcm, dstm, wm).reshape(NC, NPAD, D)
  return out[:, :N, :]


def _dotT(a, w):
  return lax.dot_general(a, w, (((1,), (1,)), ((), ())),
                         preferred_element_type=jnp.float32)


def _bn_tanh(h, gamma, beta):
  mean = jnp.mean(h, axis=0)
  c = h - mean[None, :]
  var = jnp.mean(c * c, axis=0)
  return jnp.tanh(c * (gamma / jnp.sqrt(var + 1e-5))[None, :] + beta[None, :])


def _dense1(parts, x, w_rel, b, w_root, gamma, beta):
  def body(p_ref, x_ref, wrel_ref, b_ref, wroot_ref, g_ref, be_ref, o_ref):
    agg = p_ref[0] + p_ref[1]
    h = _dotT(agg, wrel_ref[...]) + b_ref[...][None, :]
    h = h + _dotT(x_ref[...], wroot_ref[...])
    o_ref[...] = _bn_tanh(h, g_ref[...], be_ref[...])

  return pl.pallas_call(
      body, out_shape=jax.ShapeDtypeStruct((N, D), jnp.float32),
  )(parts, x, w_rel, b, w_root, gamma, beta)


def _dense2(parts, h1, w_rel, b, w_root, gamma, beta, w3_rel):
  def body(p_ref, h1_ref, wrel_ref, b_ref, wroot_ref, g_ref, be_ref,
           w3_ref, h2_ref, h2t_ref):
    agg = p_ref[0] + p_ref[1]
    h = _dotT(agg, wrel_ref[...]) + b_ref[...][None, :]
    h = h + _dotT(h1_ref[...], wroot_ref[...])
    h2 = _bn_tanh(h, g_ref[...], be_ref[...])
    h2_ref[...] = h2
    h2t_ref[...] = _dotT(h2, w3_ref[...])

  return pl.pallas_call(
      body,
      out_shape=[jax.ShapeDtypeStruct((N, HC2), jnp.float32),
                 jax.ShapeDtypeStruct((N, D), jnp.float32)],
  )(parts, h1, w_rel, b, w_root, gamma, beta, w3_rel)


def _dense3(parts, h2, b, w_root):
  def body(p_ref, h2_ref, b_ref, wroot_ref, o_ref):
    agg = p_ref[0] + p_ref[1]
    o_ref[...] = agg + b_ref[...][None, :] + _dotT(h2_ref[...], wroot_ref[...])

  return pl.pallas_call(
      body, out_shape=jax.ShapeDtypeStruct((N, D), jnp.float32),
  )(parts, h2, b, w_root)


def kernel(x, edge_index, edge_attr,
           W1_rel, b1_rel, W1_root, gamma1, beta1,
           W2_rel, b2_rel, W2_root, gamma2, beta2,
           W3_rel, b3_rel, W3_root):
  pad = E_PAD - E
  src = jnp.concatenate([edge_index[0], jnp.zeros((pad,), jnp.int32)])
  dst = jnp.concatenate([edge_index[1], jnp.zeros((pad,), jnp.int32)])
  w = jnp.concatenate([edge_attr, jnp.zeros((pad,), jnp.float32)])
  srcm = src.reshape(NW, NCHUNK, CH)
  dstm = dst.reshape(NW, NCHUNK, CH)
  wm = w.reshape(NW, EPT)

  agg1 = _sc_agg(x, srcm, dstm, wm)
  h1 = _dense1(agg1, x, W1_rel, b1_rel, W1_root, gamma1, beta1)
  agg2 = _sc_agg(h1, srcm, dstm, wm)
  h2, h2t = _dense2(agg2, h1, W2_rel, b2_rel, W2_root, gamma2, beta2, W3_rel)
  agg3 = _sc_agg(h2t, srcm, dstm, wm)
  return _dense3(agg3, h2, b3_rel, W3_root)
